# Initial kernel scaffold; baseline (speedup 1.0000x reference)
import jax
import jax.numpy as jnp
from jax.experimental import pallas as pl


def kernel(x, W1, g1, b1, m1, v1, W2, g2, b2, m2, v2, W3, g3, b3, m3, v3, fc1_w, fc1_b, fc2_w, fc2_b):
    raise NotImplementedError("write your pallas kernel here")


# speedup vs baseline: 4.3273x; 4.3273x over previous
import functools

import jax
import jax.numpy as jnp
from jax.experimental import pallas as pl
from jax.experimental.pallas import tpu as pltpu

B, N, D, K = 8, 2048, 3, 20
NUM_CLASSES = 40
TN = 256
NT = N // TN
NEG = -3.0e38


def _fused_kernel(xt_ref, xr_ref, sqr_ref, sqc_ref, at_ref, bt_ref, t1_ref,
                  w2_ref, t2_ref, w3_ref, t3_ref,
                  f1_ref, f1b_ref, f2_ref, f2b_ref,
                  out_ref, pd_scr, acc_scr):
    i = pl.program_id(1)

    xt = xt_ref[0]
    xrow = xr_ref[0]
    x0 = xrow[:, 0:1]
    x1 = xrow[:, 1:2]
    x2 = xrow[:, 2:3]
    r0 = xt[0:1, :]
    r1 = xt[1:2, :]
    r2 = xt[2:3, :]

    r0b = jnp.broadcast_to(r0, (TN, N))
    r1b = jnp.broadcast_to(r1, (TN, N))
    r2b = jnp.broadcast_to(r2, (TN, N))

    inner = -2.0 * jax.lax.dot_general(
        xrow, xt, (((1,), (0,)), ((), ())),
        precision=jax.lax.Precision.DEFAULT,
        preferred_element_type=jnp.float32)
    sqr = sqr_ref[0]
    sqc = sqc_ref[0]
    pd_scr[...] = -sqc - inner - sqr

    a0 = at_ref[0:1, :]
    a1 = at_ref[1:2, :]
    a2 = at_ref[2:3, :]
    base = (x0 * bt_ref[0:1, :] + x1 * bt_ref[1:2, :]
            + x2 * bt_ref[2:3, :] + t1_ref[...])

    iota = jax.lax.broadcasted_iota(jnp.int32, (TN, N), 1)

    def body(_, hmax):
        pd = pd_scr[...]
        m = jnp.max(pd, axis=1, keepdims=True)
        cand = jnp.where(pd >= m, iota, N)
        idx = jnp.min(cand, axis=1, keepdims=True)
        onehot = iota == idx
        nb0 = jnp.sum(jnp.where(onehot, r0b, 0.0), axis=1, keepdims=True)
        nb1 = jnp.sum(jnp.where(onehot, r1b, 0.0), axis=1, keepdims=True)
        nb2 = jnp.sum(jnp.where(onehot, r2b, 0.0), axis=1, keepdims=True)
        pd_scr[...] = jnp.where(onehot, NEG, pd)
        h = base + nb0 * a0 + nb1 * a1 + nb2 * a2
        return jnp.maximum(hmax, jnp.maximum(h, 0.0))

    hmax = jax.lax.fori_loop(0, K, body, jnp.zeros((TN, 64), jnp.float32))

    h2 = jnp.maximum(
        jnp.dot(hmax, w2_ref[...], preferred_element_type=jnp.float32)
        + t2_ref[...], 0.0)
    h3 = jnp.maximum(
        jnp.dot(h2, w3_ref[...], preferred_element_type=jnp.float32)
        + t3_ref[...], 0.0)
    tmax = jnp.max(h3, axis=0, keepdims=True)

    @pl.when(i == 0)
    def _():
        acc_scr[...] = tmax

    @pl.when(i > 0)
    def _():
        acc_scr[...] = jnp.maximum(acc_scr[...], tmax)

    @pl.when(i == NT - 1)
    def _():
        hv = acc_scr[...]
        f1 = jnp.maximum(
            jnp.dot(hv, f1_ref[...], preferred_element_type=jnp.float32)
            + f1b_ref[...], 0.0)
        out_ref[...] = (jnp.dot(f1, f2_ref[...],
                                preferred_element_type=jnp.float32)
                        + f2b_ref[...])[None]


@jax.jit
def kernel(x, W1, g1, b1, m1, v1, W2, g2, b2, m2, v2, W3, g3, b3, m3, v3,
           fc1_w, fc1_b, fc2_w, fc2_b):
    eps = 1e-5
    s1 = g1 / jnp.sqrt(v1 + eps)
    s2 = g2 / jnp.sqrt(v2 + eps)
    s3 = g3 / jnp.sqrt(v3 + eps)
    t1 = (b1 - m1 * s1)[None, :]
    t2 = (b2 - m2 * s2)[None, :]
    t3 = (b3 - m3 * s3)[None, :]
    W1a = W1[:, :3] * s1[:, None]
    W1d = (W1[:, 3:] - W1[:, :3]) * s1[:, None]
    at = W1a.T
    bt = W1d.T
    w2 = (W2 * s2[:, None]).T
    w3 = (W3 * s3[:, None]).T
    f1 = fc1_w.T
    f1b = fc1_b[None, :]
    f2 = jnp.zeros((128, 128), jnp.float32).at[:, :NUM_CLASSES].set(fc2_w.T)
    f2b = jnp.zeros((1, 128), jnp.float32).at[:, :NUM_CLASSES].set(fc2_b)

    xt = jnp.transpose(x, (0, 2, 1))
    xx = jnp.sum(x ** 2, axis=2, keepdims=True)
    sqr_in = jnp.transpose(xx, (0, 2, 1))
    sqc_in = xx

    out = pl.pallas_call(
        _fused_kernel,
        grid=(B, NT),
        in_specs=[
            pl.BlockSpec((1, D, N), lambda b, i: (b, 0, 0)),
            pl.BlockSpec((1, TN, D), lambda b, i: (b, i, 0)),
            pl.BlockSpec((1, 1, N), lambda b, i: (b, 0, 0)),
            pl.BlockSpec((1, TN, 1), lambda b, i: (b, i, 0)),
            pl.BlockSpec((D, 64), lambda b, i: (0, 0)),
            pl.BlockSpec((D, 64), lambda b, i: (0, 0)),
            pl.BlockSpec((1, 64), lambda b, i: (0, 0)),
            pl.BlockSpec((64, 128), lambda b, i: (0, 0)),
            pl.BlockSpec((1, 128), lambda b, i: (0, 0)),
            pl.BlockSpec((128, 256), lambda b, i: (0, 0)),
            pl.BlockSpec((1, 256), lambda b, i: (0, 0)),
            pl.BlockSpec((256, 128), lambda b, i: (0, 0)),
            pl.BlockSpec((1, 128), lambda b, i: (0, 0)),
            pl.BlockSpec((128, 128), lambda b, i: (0, 0)),
            pl.BlockSpec((1, 128), lambda b, i: (0, 0)),
        ],
        out_specs=pl.BlockSpec((1, 1, 128), lambda b, i: (b, 0, 0)),
        out_shape=jax.ShapeDtypeStruct((B, 1, 128), jnp.float32),
        scratch_shapes=[
            pltpu.VMEM((TN, N), jnp.float32),
            pltpu.VMEM((1, 256), jnp.float32),
        ],
        compiler_params=pltpu.CompilerParams(
            dimension_semantics=("parallel", "arbitrary"),
        ),
    )(xt, x, sqr_in, sqc_in, at, bt, t1, w2, t2, w3, t3, f1, f1b, f2, f2b)
    return out[:, 0, :NUM_CLASSES]


# speedup vs baseline: 5.6054x; 1.2954x over previous
import functools

import jax
import jax.numpy as jnp
from jax.experimental import pallas as pl
from jax.experimental.pallas import tpu as pltpu

B, N, D, K = 8, 2048, 3, 20
NUM_CLASSES = 40
TN = 256
NT = N // TN
NEG = -3.0e38


def _fused_kernel(xt_ref, xr_ref, xf_ref, sqr_ref, sqc_ref, at_ref, bt_ref,
                  t1_ref, w2_ref, t2_ref, w3_ref, t3_ref,
                  f1_ref, f1b_ref, f2_ref, f2b_ref,
                  out_ref, pd_scr, acc_scr):
    i = pl.program_id(1)

    xt = xt_ref[0]
    xrow = xr_ref[0]
    x0 = xrow[:, 0:1]
    x1 = xrow[:, 1:2]
    x2 = xrow[:, 2:3]

    inner = -2.0 * jax.lax.dot_general(
        xrow, xt, (((1,), (0,)), ((), ())),
        precision=jax.lax.Precision.DEFAULT,
        preferred_element_type=jnp.float32)
    sqr = sqr_ref[0]
    sqc = sqc_ref[0]
    pd_scr[...] = -sqc - inner - sqr

    G = jnp.dot(xf_ref[0], at_ref[...],
                preferred_element_type=jnp.float32)
    base = (x0 * bt_ref[0:1, :] + x1 * bt_ref[1:2, :]
            + x2 * bt_ref[2:3, :] + t1_ref[...])

    iota = jax.lax.broadcasted_iota(jnp.int32, (TN, N), 1)

    def body(_, hmax):
        pd = pd_scr[...]
        m = jnp.max(pd, axis=1, keepdims=True)
        cand = jnp.where(pd >= m, iota, N)
        idx = jnp.min(cand, axis=1, keepdims=True)
        onehot = iota == idx
        ohf = jnp.where(onehot, 1.0, 0.0)
        pd_scr[...] = jnp.where(onehot, NEG, pd)
        h = base + jnp.dot(ohf, G, preferred_element_type=jnp.float32)
        return jnp.maximum(hmax, jnp.maximum(h, 0.0))

    hmax = jax.lax.fori_loop(0, K, body, jnp.zeros((TN, 64), jnp.float32))

    h2 = jnp.maximum(
        jnp.dot(hmax, w2_ref[...], preferred_element_type=jnp.float32)
        + t2_ref[...], 0.0)
    h3 = jnp.maximum(
        jnp.dot(h2, w3_ref[...], preferred_element_type=jnp.float32)
        + t3_ref[...], 0.0)
    tmax = jnp.max(h3, axis=0, keepdims=True)

    @pl.when(i == 0)
    def _():
        acc_scr[...] = tmax

    @pl.when(i > 0)
    def _():
        acc_scr[...] = jnp.maximum(acc_scr[...], tmax)

    @pl.when(i == NT - 1)
    def _():
        hv = acc_scr[...]
        f1 = jnp.maximum(
            jnp.dot(hv, f1_ref[...], preferred_element_type=jnp.float32)
            + f1b_ref[...], 0.0)
        out_ref[...] = (jnp.dot(f1, f2_ref[...],
                                preferred_element_type=jnp.float32)
                        + f2b_ref[...])[None]


@jax.jit
def kernel(x, W1, g1, b1, m1, v1, W2, g2, b2, m2, v2, W3, g3, b3, m3, v3,
           fc1_w, fc1_b, fc2_w, fc2_b):
    eps = 1e-5
    s1 = g1 / jnp.sqrt(v1 + eps)
    s2 = g2 / jnp.sqrt(v2 + eps)
    s3 = g3 / jnp.sqrt(v3 + eps)
    t1 = (b1 - m1 * s1)[None, :]
    t2 = (b2 - m2 * s2)[None, :]
    t3 = (b3 - m3 * s3)[None, :]
    W1a = W1[:, :3] * s1[:, None]
    W1d = (W1[:, 3:] - W1[:, :3]) * s1[:, None]
    at = W1a.T
    bt = W1d.T
    w2 = (W2 * s2[:, None]).T
    w3 = (W3 * s3[:, None]).T
    f1 = fc1_w.T
    f1b = fc1_b[None, :]
    f2 = jnp.zeros((128, 128), jnp.float32).at[:, :NUM_CLASSES].set(fc2_w.T)
    f2b = jnp.zeros((1, 128), jnp.float32).at[:, :NUM_CLASSES].set(fc2_b)

    xt = jnp.transpose(x, (0, 2, 1))
    xx = jnp.sum(x ** 2, axis=2, keepdims=True)
    sqr_in = jnp.transpose(xx, (0, 2, 1))
    sqc_in = xx

    out = pl.pallas_call(
        _fused_kernel,
        grid=(B, NT),
        in_specs=[
            pl.BlockSpec((1, D, N), lambda b, i: (b, 0, 0)),
            pl.BlockSpec((1, TN, D), lambda b, i: (b, i, 0)),
            pl.BlockSpec((1, N, D), lambda b, i: (b, 0, 0)),
            pl.BlockSpec((1, 1, N), lambda b, i: (b, 0, 0)),
            pl.BlockSpec((1, TN, 1), lambda b, i: (b, i, 0)),
            pl.BlockSpec((D, 64), lambda b, i: (0, 0)),
            pl.BlockSpec((D, 64), lambda b, i: (0, 0)),
            pl.BlockSpec((1, 64), lambda b, i: (0, 0)),
            pl.BlockSpec((64, 128), lambda b, i: (0, 0)),
            pl.BlockSpec((1, 128), lambda b, i: (0, 0)),
            pl.BlockSpec((128, 256), lambda b, i: (0, 0)),
            pl.BlockSpec((1, 256), lambda b, i: (0, 0)),
            pl.BlockSpec((256, 128), lambda b, i: (0, 0)),
            pl.BlockSpec((1, 128), lambda b, i: (0, 0)),
            pl.BlockSpec((128, 128), lambda b, i: (0, 0)),
            pl.BlockSpec((1, 128), lambda b, i: (0, 0)),
        ],
        out_specs=pl.BlockSpec((1, 1, 128), lambda b, i: (b, 0, 0)),
        out_shape=jax.ShapeDtypeStruct((B, 1, 128), jnp.float32),
        scratch_shapes=[
            pltpu.VMEM((TN, N), jnp.float32),
            pltpu.VMEM((1, 256), jnp.float32),
        ],
        compiler_params=pltpu.CompilerParams(
            dimension_semantics=("parallel", "arbitrary"),
        ),
    )(xt, x, x, sqr_in, sqc_in, at, bt, t1, w2, t2, w3, t3, f1, f1b, f2, f2b)
    return out[:, 0, :NUM_CLASSES]


# speedup vs baseline: 6.5656x; 1.1713x over previous
import functools

import jax
import jax.numpy as jnp
from jax.experimental import pallas as pl
from jax.experimental.pallas import tpu as pltpu

B, N, D, K = 8, 2048, 3, 20
NUM_CLASSES = 40
TN = 256
NT = N // TN
NEG = -3.0e38


def _fused_kernel(xt_ref, xr_ref, xf_ref, sqr_ref, sqc_ref, at_ref, bt_ref,
                  t1_ref, w2_ref, t2_ref, w3_ref, t3_ref,
                  f1_ref, f1b_ref, f2_ref, f2b_ref,
                  out_ref, pd_scr, acc_scr):
    i = pl.program_id(1)

    xt = xt_ref[0]
    xrow = xr_ref[0]
    x0 = xrow[:, 0:1]
    x1 = xrow[:, 1:2]
    x2 = xrow[:, 2:3]

    inner = -2.0 * jax.lax.dot_general(
        xrow, xt, (((1,), (0,)), ((), ())),
        precision=jax.lax.Precision.DEFAULT,
        preferred_element_type=jnp.float32)
    sqr = sqr_ref[0]
    sqc = sqc_ref[0]
    pd_scr[...] = -sqc - inner - sqr

    G = jnp.dot(xf_ref[0], at_ref[...],
                preferred_element_type=jnp.float32)
    base = (x0 * bt_ref[0:1, :] + x1 * bt_ref[1:2, :]
            + x2 * bt_ref[2:3, :] + t1_ref[...])

    iota = jax.lax.broadcasted_iota(jnp.int32, (TN, N), 1)

    def _pick(pd, hmax):
        m = jnp.max(pd, axis=1, keepdims=True)
        cand = jnp.where(pd >= m, iota, N)
        idx = jnp.min(cand, axis=1, keepdims=True)
        onehot = iota == idx
        ohf = jnp.where(onehot, 1.0, 0.0)
        pdn = jnp.where(onehot, NEG, pd)
        h = base + jnp.dot(ohf, G, preferred_element_type=jnp.float32)
        return pdn, jnp.maximum(hmax, jnp.maximum(h, 0.0))

    def body(_, hmax):
        pd, hmax = _pick(pd_scr[...], hmax)
        pd, hmax = _pick(pd, hmax)
        pd_scr[...] = pd
        return hmax

    hmax = jax.lax.fori_loop(0, K // 2, body,
                             jnp.zeros((TN, 64), jnp.float32))

    h2 = jnp.maximum(
        jnp.dot(hmax, w2_ref[...], preferred_element_type=jnp.float32)
        + t2_ref[...], 0.0)
    h3 = jnp.maximum(
        jnp.dot(h2, w3_ref[...], preferred_element_type=jnp.float32)
        + t3_ref[...], 0.0)
    tmax = jnp.max(h3, axis=0, keepdims=True)

    @pl.when(i == 0)
    def _():
        acc_scr[...] = tmax

    @pl.when(i > 0)
    def _():
        acc_scr[...] = jnp.maximum(acc_scr[...], tmax)

    @pl.when(i == NT - 1)
    def _():
        hv = acc_scr[...]
        f1 = jnp.maximum(
            jnp.dot(hv, f1_ref[...], preferred_element_type=jnp.float32)
            + f1b_ref[...], 0.0)
        out_ref[...] = (jnp.dot(f1, f2_ref[...],
                                preferred_element_type=jnp.float32)
                        + f2b_ref[...])[None]


@jax.jit
def kernel(x, W1, g1, b1, m1, v1, W2, g2, b2, m2, v2, W3, g3, b3, m3, v3,
           fc1_w, fc1_b, fc2_w, fc2_b):
    eps = 1e-5
    s1 = g1 / jnp.sqrt(v1 + eps)
    s2 = g2 / jnp.sqrt(v2 + eps)
    s3 = g3 / jnp.sqrt(v3 + eps)
    t1 = (b1 - m1 * s1)[None, :]
    t2 = (b2 - m2 * s2)[None, :]
    t3 = (b3 - m3 * s3)[None, :]
    W1a = W1[:, :3] * s1[:, None]
    W1d = (W1[:, 3:] - W1[:, :3]) * s1[:, None]
    at = W1a.T
    bt = W1d.T
    w2 = (W2 * s2[:, None]).T
    w3 = (W3 * s3[:, None]).T
    f1 = fc1_w.T
    f1b = fc1_b[None, :]
    f2 = jnp.zeros((128, 128), jnp.float32).at[:, :NUM_CLASSES].set(fc2_w.T)
    f2b = jnp.zeros((1, 128), jnp.float32).at[:, :NUM_CLASSES].set(fc2_b)

    xt = jnp.transpose(x, (0, 2, 1))
    xx = jnp.sum(x ** 2, axis=2, keepdims=True)
    sqr_in = jnp.transpose(xx, (0, 2, 1))
    sqc_in = xx

    out = pl.pallas_call(
        _fused_kernel,
        grid=(B, NT),
        in_specs=[
            pl.BlockSpec((1, D, N), lambda b, i: (b, 0, 0)),
            pl.BlockSpec((1, TN, D), lambda b, i: (b, i, 0)),
            pl.BlockSpec((1, N, D), lambda b, i: (b, 0, 0)),
            pl.BlockSpec((1, 1, N), lambda b, i: (b, 0, 0)),
            pl.BlockSpec((1, TN, 1), lambda b, i: (b, i, 0)),
            pl.BlockSpec((D, 64), lambda b, i: (0, 0)),
            pl.BlockSpec((D, 64), lambda b, i: (0, 0)),
            pl.BlockSpec((1, 64), lambda b, i: (0, 0)),
            pl.BlockSpec((64, 128), lambda b, i: (0, 0)),
            pl.BlockSpec((1, 128), lambda b, i: (0, 0)),
            pl.BlockSpec((128, 256), lambda b, i: (0, 0)),
            pl.BlockSpec((1, 256), lambda b, i: (0, 0)),
            pl.BlockSpec((256, 128), lambda b, i: (0, 0)),
            pl.BlockSpec((1, 128), lambda b, i: (0, 0)),
            pl.BlockSpec((128, 128), lambda b, i: (0, 0)),
            pl.BlockSpec((1, 128), lambda b, i: (0, 0)),
        ],
        out_specs=pl.BlockSpec((1, 1, 128), lambda b, i: (b, 0, 0)),
        out_shape=jax.ShapeDtypeStruct((B, 1, 128), jnp.float32),
        scratch_shapes=[
            pltpu.VMEM((TN, N), jnp.float32),
            pltpu.VMEM((1, 256), jnp.float32),
        ],
        compiler_params=pltpu.CompilerParams(
            dimension_semantics=("parallel", "arbitrary"),
        ),
    )(xt, x, x, sqr_in, sqc_in, at, bt, t1, w2, t2, w3, t3, f1, f1b, f2, f2b)
    return out[:, 0, :NUM_CLASSES]


# speedup vs baseline: 7.5033x; 1.1428x over previous
import jax
import jax.numpy as jnp
from jax.experimental import pallas as pl
from jax.experimental.pallas import tpu as pltpu

B, N, D, K = 8, 2048, 3, 20
NUM_CLASSES = 40
TN = 512
NT = N // TN
NEG = -3.0e38


def _fused_kernel(xt_ref, xr_ref, xf_ref, sqr_ref, sqc_ref, at_ref, bt_ref,
                  t1_ref, w2_ref, t2_ref, w3_ref, t3_ref,
                  f1_ref, f1b_ref, f2_ref, f2b_ref,
                  out_ref, pd_scr, acc_scr):
    i = pl.program_id(1)

    xt = xt_ref[0]
    xrow = xr_ref[0]
    x0 = xrow[:, 0:1]
    x1 = xrow[:, 1:2]
    x2 = xrow[:, 2:3]

    inner = -2.0 * jax.lax.dot_general(
        xrow, xt, (((1,), (0,)), ((), ())),
        precision=jax.lax.Precision.DEFAULT,
        preferred_element_type=jnp.float32)
    sqr = sqr_ref[0]
    sqc = sqc_ref[0]
    pd_scr[...] = -sqc - inner - sqr

    G = jnp.dot(xf_ref[0], at_ref[...],
                preferred_element_type=jnp.float32)
    base = (x0 * bt_ref[0:1, :] + x1 * bt_ref[1:2, :]
            + x2 * bt_ref[2:3, :] + t1_ref[...])

    iota = jax.lax.broadcasted_iota(jnp.int32, (TN, N), 1)

    def _pick(pd, hmax):
        m = jnp.max(pd, axis=1, keepdims=True)
        cand = jnp.where(pd >= m, iota, N)
        idx = jnp.min(cand, axis=1, keepdims=True)
        onehot = iota == idx
        ohf = jnp.where(onehot, 1.0, 0.0)
        pdn = jnp.where(onehot, NEG, pd)
        h = base + jnp.dot(ohf, G, preferred_element_type=jnp.float32)
        return pdn, jnp.maximum(hmax, jnp.maximum(h, 0.0))

    def body(_, hmax):
        pd, hmax = _pick(pd_scr[...], hmax)
        pd, hmax = _pick(pd, hmax)
        pd, hmax = _pick(pd, hmax)
        pd, hmax = _pick(pd, hmax)
        pd_scr[...] = pd
        return hmax

    hmax = jax.lax.fori_loop(0, K // 4, body,
                             jnp.zeros((TN, 64), jnp.float32))

    h2 = jnp.maximum(
        jnp.dot(hmax, w2_ref[...], preferred_element_type=jnp.float32)
        + t2_ref[...], 0.0)
    h3 = jnp.maximum(
        jnp.dot(h2, w3_ref[...], preferred_element_type=jnp.float32)
        + t3_ref[...], 0.0)
    tmax = jnp.max(h3, axis=0, keepdims=True)

    @pl.when(i == 0)
    def _():
        acc_scr[...] = tmax

    @pl.when(i > 0)
    def _():
        acc_scr[...] = jnp.maximum(acc_scr[...], tmax)

    @pl.when(i == NT - 1)
    def _():
        hv = acc_scr[...]
        f1 = jnp.maximum(
            jnp.dot(hv, f1_ref[...], preferred_element_type=jnp.float32)
            + f1b_ref[...], 0.0)
        out_ref[...] = (jnp.dot(f1, f2_ref[...],
                                preferred_element_type=jnp.float32)
                        + f2b_ref[...])[None]


@jax.jit
def kernel(x, W1, g1, b1, m1, v1, W2, g2, b2, m2, v2, W3, g3, b3, m3, v3,
           fc1_w, fc1_b, fc2_w, fc2_b):
    eps = 1e-5
    s1 = g1 / jnp.sqrt(v1 + eps)
    s2 = g2 / jnp.sqrt(v2 + eps)
    s3 = g3 / jnp.sqrt(v3 + eps)
    t1 = (b1 - m1 * s1)[None, :]
    t2 = (b2 - m2 * s2)[None, :]
    t3 = (b3 - m3 * s3)[None, :]
    W1a = W1[:, :3] * s1[:, None]
    W1d = (W1[:, 3:] - W1[:, :3]) * s1[:, None]
    at = W1a.T
    bt = W1d.T
    w2 = (W2 * s2[:, None]).T
    w3 = (W3 * s3[:, None]).T
    f1 = fc1_w.T
    f1b = fc1_b[None, :]
    f2 = jnp.zeros((128, 128), jnp.float32).at[:, :NUM_CLASSES].set(fc2_w.T)
    f2b = jnp.zeros((1, 128), jnp.float32).at[:, :NUM_CLASSES].set(fc2_b)

    xt = jnp.transpose(x, (0, 2, 1))
    xx = jnp.sum(x ** 2, axis=2, keepdims=True)
    sqr_in = jnp.transpose(xx, (0, 2, 1))
    sqc_in = xx

    out = pl.pallas_call(
        _fused_kernel,
        grid=(B, NT),
        in_specs=[
            pl.BlockSpec((1, D, N), lambda b, i: (b, 0, 0)),
            pl.BlockSpec((1, TN, D), lambda b, i: (b, i, 0)),
            pl.BlockSpec((1, N, D), lambda b, i: (b, 0, 0)),
            pl.BlockSpec((1, 1, N), lambda b, i: (b, 0, 0)),
            pl.BlockSpec((1, TN, 1), lambda b, i: (b, i, 0)),
            pl.BlockSpec((D, 64), lambda b, i: (0, 0)),
            pl.BlockSpec((D, 64), lambda b, i: (0, 0)),
            pl.BlockSpec((1, 64), lambda b, i: (0, 0)),
            pl.BlockSpec((64, 128), lambda b, i: (0, 0)),
            pl.BlockSpec((1, 128), lambda b, i: (0, 0)),
            pl.BlockSpec((128, 256), lambda b, i: (0, 0)),
            pl.BlockSpec((1, 256), lambda b, i: (0, 0)),
            pl.BlockSpec((256, 128), lambda b, i: (0, 0)),
            pl.BlockSpec((1, 128), lambda b, i: (0, 0)),
            pl.BlockSpec((128, 128), lambda b, i: (0, 0)),
            pl.BlockSpec((1, 128), lambda b, i: (0, 0)),
        ],
        out_specs=pl.BlockSpec((1, 1, 128), lambda b, i: (b, 0, 0)),
        out_shape=jax.ShapeDtypeStruct((B, 1, 128), jnp.float32),
        scratch_shapes=[
            pltpu.VMEM((TN, N), jnp.float32),
            pltpu.VMEM((1, 256), jnp.float32),
        ],
        compiler_params=pltpu.CompilerParams(
            dimension_semantics=("parallel", "arbitrary"),
        ),
    )(xt, x, x, sqr_in, sqc_in, at, bt, t1, w2, t2, w3, t3, f1, f1b, f2, f2b)
    return out[:, 0, :NUM_CLASSES]
